# Initial kernel scaffold; baseline (speedup 1.0000x reference)
import functools

import jax
import jax.numpy as jnp
from jax import lax
from jax.experimental import pallas as pl
from jax.experimental.pallas import tpu as pltpu
from jax.experimental.pallas import tpu_sc as plsc

_V = 1000
_D = 200
_B = 1024
_L = 200
_N = _B * _L
_NC = 2
_NS = 16
_NW = _NC * _NS
_PER_W = _N // _NW
_CHUNK = 128
_NCHUNK = _PER_W // _CHUNK
_NBUF = 2
_NGROUP = _NCHUNK // _NBUF

_mesh = plsc.VectorSubcoreMesh(core_axis_name="c", subcore_axis_name="s")


@functools.partial(
    pl.kernel,
    mesh=_mesh,
    out_type=jax.ShapeDtypeStruct((_N, _D), jnp.float32),
    scratch_types=(
        [pltpu.VMEM((_NCHUNK, _CHUNK), jnp.int32)]
        + [pltpu.VMEM((_CHUNK, _D), jnp.float32) for _ in range(_NBUF)]
        + [pltpu.SemaphoreType.DMA for _ in range(2 * _NBUF)]
    ),
)
def _embed(idx_hbm, table_hbm, out_hbm, idx_v, *rest):
    rows = rest[:_NBUF]
    gsem = rest[_NBUF:2 * _NBUF]
    ssem = rest[2 * _NBUF:]

    wid = lax.axis_index("s") * _NC + lax.axis_index("c")
    base = wid * _PER_W

    pltpu.sync_copy(idx_hbm.at[wid], idx_v)

    def gather(c, b):
        return pltpu.make_async_copy(
            table_hbm.at[idx_v.at[c]], rows[b], gsem[b])

    def scatter(c, b):
        return pltpu.make_async_copy(
            rows[b], out_hbm.at[pl.ds(base + c * _CHUNK, _CHUNK)], ssem[b])

    for b in range(_NBUF):
        gather(b, b).start()

    def group_body(g, carry):
        for b in range(_NBUF):
            c = g * _NBUF + b
            gather(c, b).wait()
            scatter(c, b).start()
        for b in range(_NBUF):
            c = g * _NBUF + b
            scatter(c, b).wait()
            gather((g + 1) * _NBUF + b, b).start()
        return carry

    lax.fori_loop(0, _NGROUP - 1, group_body, 0)

    g_last = _NGROUP - 1
    for b in range(_NBUF):
        c = g_last * _NBUF + b
        gather(c, b).wait()
        scatter(c, b).start()
    for b in range(_NBUF):
        scatter(g_last * _NBUF + b, b).wait()


def kernel(inputs, table):
    idx = inputs.reshape(_NW, _NCHUNK, _CHUNK)
    out = _embed(idx, table)
    return out.reshape(_B, _L * _D)


# speedup vs baseline: 3.2444x; 3.2444x over previous
import functools

import jax
import jax.numpy as jnp
from jax import lax
from jax.experimental import pallas as pl
from jax.experimental.pallas import tpu as pltpu
from jax.experimental.pallas import tpu_sc as plsc

_V = 1000
_D = 200
_B = 1024
_L = 200
_N = _B * _L
_NC = 2
_NS = 16
_NW = _NC * _NS
_PER_W = _N // _NW
_CHUNK = 64
_NCHUNK = _PER_W // _CHUNK
_NBUF = 4
_NGROUP = _NCHUNK // _NBUF

_mesh = plsc.VectorSubcoreMesh(core_axis_name="c", subcore_axis_name="s")


@functools.partial(
    pl.kernel,
    mesh=_mesh,
    out_type=jax.ShapeDtypeStruct((_N, _D), jnp.float32),
    compiler_params=pltpu.CompilerParams(use_tc_tiling_on_sc=False),
    scratch_types=(
        [pltpu.VMEM((_NCHUNK, _CHUNK), jnp.int32),
         pltpu.VMEM_SHARED((_V, _D), jnp.float32)]
        + [pltpu.VMEM((_CHUNK, _D), jnp.float32) for _ in range(_NBUF)]
        + [pltpu.SemaphoreType.DMA for _ in range(2 * _NBUF)]
    ),
)
def _embed(idx_hbm, table_hbm, out_hbm, idx_v, table_sp, *rest):
    rows = rest[:_NBUF]
    gsem = rest[_NBUF:2 * _NBUF]
    ssem = rest[2 * _NBUF:]

    sid = lax.axis_index("s")
    wid = sid * _NC + lax.axis_index("c")
    base = wid * _PER_W

    @pl.when(sid == 0)
    def _():
        pltpu.sync_copy(table_hbm, table_sp)

    pltpu.sync_copy(idx_hbm.at[wid], idx_v)
    plsc.subcore_barrier()

    def gather(c, b):
        return pltpu.make_async_copy(
            table_sp.at[idx_v.at[c]], rows[b], gsem[b])

    def scatter(c, b):
        return pltpu.make_async_copy(
            rows[b], out_hbm.at[pl.ds(base + c * _CHUNK, _CHUNK)], ssem[b])

    for b in range(_NBUF):
        gather(b, b).start()

    def group_body(g, carry):
        for b in range(_NBUF):
            c = g * _NBUF + b
            gather(c, b).wait()
            scatter(c, b).start()
        for b in range(_NBUF):
            c = g * _NBUF + b
            scatter(c, b).wait()
            gather((g + 1) * _NBUF + b, b).start()
        return carry

    lax.fori_loop(0, _NGROUP - 1, group_body, 0)

    g_last = _NGROUP - 1
    for b in range(_NBUF):
        c = g_last * _NBUF + b
        gather(c, b).wait()
        scatter(c, b).start()
    for b in range(_NBUF):
        scatter(g_last * _NBUF + b, b).wait()


def kernel(inputs, table):
    idx = inputs.reshape(_NW, _NCHUNK, _CHUNK)
    out = _embed(idx, table)
    return out.reshape(_B, _L * _D)
